# baseline (speedup 1.0000x reference)
import functools

import jax
import jax.numpy as jnp
from jax import lax
from jax.experimental import pallas as pl
from jax.experimental.pallas import tpu as pltpu
from jax.experimental.pallas import tpu_sc as plsc

_N = 4096
_NC = 2
_NS = 16
_NW = _NC * _NS
_RC = 4
_THRESHOLD = 1.0

_VELTKAMP_C = 65537.0


def _round_bf16(v):
    p = _VELTKAMP_C * v
    q = v - p
    return p + q


def _make_sc_spikes(n_rows, row_base=0):
    rpw = n_rows // _NW
    assert n_rows % _NW == 0 and rpw % (4 * _RC) == 0
    n_chunks = rpw // _RC
    mesh = plsc.VectorSubcoreMesh(
        core_axis_name="c", subcore_axis_name="s", num_cores=_NC, num_subcores=_NS
    )

    @functools.partial(
        pl.kernel,
        out_type=jax.ShapeDtypeStruct((n_rows,), jnp.float32),
        mesh=mesh,
        scratch_types=[
            pltpu.VMEM((_N,), jnp.float32),
            pltpu.VMEM((2, _RC, _N), jnp.float32),
            pltpu.VMEM((2, _RC, _N), jnp.float32),
            pltpu.VMEM((rpw,), jnp.float32),
            pltpu.SemaphoreType.DMA((2,)),
            pltpu.SemaphoreType.DMA((2,)),
        ],
    )
    def sc_spikes(x_hbm, w_hbm, m_hbm, o_hbm, x_v, w_v, m_v, o_v, w_sem, m_sem):
        wid = lax.axis_index("s") * _NC + lax.axis_index("c")
        out0 = wid * rpw
        row0 = row_base + out0
        pltpu.sync_copy(x_hbm, x_v)

        def round_x(i, _):
            col = 16 * i
            x_v[pl.ds(col, 16)] = _round_bf16(x_v[pl.ds(col, 16)])
            return ()

        lax.fori_loop(0, _N // 16, round_x, (), unroll=8)

        def start(c, slot):
            pltpu.async_copy(
                w_hbm.at[pl.ds(row0 + c * _RC, _RC)], w_v.at[slot], w_sem.at[slot]
            )
            pltpu.async_copy(
                m_hbm.at[pl.ds(row0 + c * _RC, _RC)], m_v.at[slot], m_sem.at[slot]
            )

        lane = lax.broadcasted_iota(jnp.int32, (16,), 0)
        start(0, 0)
        start(1, 1)

        def pair_body(g, ovec):
            for b in range(2):
                c = 2 * g + b
                pltpu.make_async_copy(
                    w_hbm.at[pl.ds(0, _RC)], w_v.at[b], w_sem.at[b]
                ).wait()
                pltpu.make_async_copy(
                    m_hbm.at[pl.ds(0, _RC)], m_v.at[b], m_sem.at[b]
                ).wait()

                def k_body(k, accs):
                    col = 32 * k
                    xa = x_v[pl.ds(col, 16)]
                    xb = x_v[pl.ds(col + 16, 16)]
                    out = []
                    for r in range(_RC):
                        wa = w_v[b, r, pl.ds(col, 16)] * m_v[b, r, pl.ds(col, 16)]
                        wb = (
                            w_v[b, r, pl.ds(col + 16, 16)]
                            * m_v[b, r, pl.ds(col + 16, 16)]
                        )
                        out.append(
                            accs[r] + _round_bf16(wa) * xa + _round_bf16(wb) * xb
                        )
                    return tuple(out)

                accs = lax.fori_loop(
                    0,
                    _N // 32,
                    k_body,
                    tuple(jnp.zeros((16,), jnp.float32) for _ in range(_RC)),
                    unroll=2,
                )

                @pl.when(c + 2 < n_chunks)
                def _():
                    start_row = row0 + (c + 2) * _RC
                    pltpu.async_copy(
                        w_hbm.at[pl.ds(start_row, _RC)], w_v.at[b], w_sem.at[b]
                    )
                    pltpu.async_copy(
                        m_hbm.at[pl.ds(start_row, _RC)], m_v.at[b], m_sem.at[b]
                    )

                for r in range(_RC):
                    s = accs[r][0]
                    for i in range(1, 16):
                        s = s + accs[r][i]
                    spike = (s - _THRESHOLD >= 0.0).astype(jnp.float32)
                    lanepos = (c % 4) * _RC + r
                    ovec = jnp.where(lane == lanepos, spike, ovec)

            @pl.when(g % 2 == 1)
            def _():
                o_v[pl.ds((g // 2) * 16, 16)] = ovec

            return jnp.where(g % 2 == 1, jnp.zeros((16,), jnp.float32), ovec)

        lax.fori_loop(0, n_chunks // 2, pair_body, jnp.zeros((16,), jnp.float32))
        pltpu.sync_copy(o_v, o_hbm.at[pl.ds(out0, rpw)])

    return sc_spikes


_TC_BM = 256


def _tc_spike_kernel(x_hbm, w_ref, m_ref, o_ref, xs_ref, sem):
    @pl.when(pl.program_id(0) == 0)
    def _():
        cp = pltpu.make_async_copy(x_hbm, xs_ref, sem)
        cp.start()
        cp.wait()

    wm = (w_ref[...] * m_ref[...]).astype(jnp.bfloat16)
    xv = xs_ref[...].astype(jnp.bfloat16)
    acc = jax.lax.dot_general(
        wm, xv, (((1,), (0,)), ((), ())), preferred_element_type=jnp.float32
    )
    o_ref[...] = (acc - _THRESHOLD >= 0.0).astype(jnp.float32)


def _tc_spikes(x, weight, weight_mask, n_rows):
    return pl.pallas_call(
        _tc_spike_kernel,
        grid=(n_rows // _TC_BM,),
        in_specs=[
            pl.BlockSpec(memory_space=pl.ANY),
            pl.BlockSpec((_TC_BM, _N), lambda i: (i, 0)),
            pl.BlockSpec((_TC_BM, _N), lambda i: (i, 0)),
        ],
        out_specs=pl.BlockSpec((_TC_BM,), lambda i: (i,)),
        out_shape=jax.ShapeDtypeStruct((n_rows,), jnp.float32),
        scratch_shapes=[
            pltpu.VMEM((_N,), jnp.float32),
            pltpu.SemaphoreType.DMA,
        ],
    )(x, weight, weight_mask)


_MR = 64
_NBUF = 6


def _tc3_kernel(x_hbm, w_hbm, m_hbm, o_ref, xs_ref, wbuf, mbuf, xsem, wsem, msem):
    cpx = pltpu.make_async_copy(x_hbm, xs_ref, xsem)
    cpx.start()

    def start(c, slot):
        pltpu.make_async_copy(
            w_hbm.at[pl.ds(c * _MR, _MR)], wbuf.at[slot], wsem.at[slot]
        ).start()
        pltpu.make_async_copy(
            m_hbm.at[pl.ds(c * _MR, _MR)], mbuf.at[slot], msem.at[slot]
        ).start()

    n_chunks = _N // _MR
    for s in range(_NBUF):
        start(s, s)
    cpx.wait()
    xv = xs_ref[...].astype(jnp.bfloat16)
    for c in range(n_chunks):
        slot = c % _NBUF
        pltpu.make_async_copy(
            w_hbm.at[pl.ds(c * _MR, _MR)], wbuf.at[slot], wsem.at[slot]
        ).wait()
        pltpu.make_async_copy(
            m_hbm.at[pl.ds(c * _MR, _MR)], mbuf.at[slot], msem.at[slot]
        ).wait()
        wm = (wbuf[slot] * mbuf[slot]).astype(jnp.bfloat16)
        acc = jax.lax.dot_general(
            wm, xv, (((1,), (0,)), ((), ())), preferred_element_type=jnp.float32
        )
        o_ref[pl.ds(c * _MR, _MR)] = (acc - _THRESHOLD >= 0.0).astype(jnp.float32)
        if c + _NBUF < n_chunks:
            start(c + _NBUF, slot)


def _tc3_spikes(x, weight, weight_mask):
    return pl.pallas_call(
        _tc3_kernel,
        in_specs=[
            pl.BlockSpec(memory_space=pl.ANY),
            pl.BlockSpec(memory_space=pl.ANY),
            pl.BlockSpec(memory_space=pl.ANY),
        ],
        out_shape=jax.ShapeDtypeStruct((_N,), jnp.float32),
        scratch_shapes=[
            pltpu.VMEM((_N,), jnp.float32),
            pltpu.VMEM((_NBUF, _MR, _N), jnp.float32),
            pltpu.VMEM((_NBUF, _MR, _N), jnp.float32),
            pltpu.SemaphoreType.DMA,
            pltpu.SemaphoreType.DMA((_NBUF,)),
            pltpu.SemaphoreType.DMA((_NBUF,)),
        ],
    )(x, weight, weight_mask)


def kernel(x, weight, weight_mask):
    return _tc3_spikes(x, weight, weight_mask)


# speedup vs baseline: 1.0294x; 1.0294x over previous
import functools

import jax
import jax.numpy as jnp
from jax import lax
from jax.experimental import pallas as pl
from jax.experimental.pallas import tpu as pltpu
from jax.experimental.pallas import tpu_sc as plsc

_N = 4096
_NC = 2
_NS = 16
_NW = _NC * _NS
_RC = 4
_THRESHOLD = 1.0

_VELTKAMP_C = 65537.0


def _round_bf16(v):
    p = _VELTKAMP_C * v
    q = v - p
    return p + q


def _make_sc_spikes(n_rows, row_base=0):
    rpw = n_rows // _NW
    assert n_rows % _NW == 0 and rpw % (4 * _RC) == 0
    n_chunks = rpw // _RC
    mesh = plsc.VectorSubcoreMesh(
        core_axis_name="c", subcore_axis_name="s", num_cores=_NC, num_subcores=_NS
    )

    @functools.partial(
        pl.kernel,
        out_type=jax.ShapeDtypeStruct((n_rows,), jnp.float32),
        mesh=mesh,
        scratch_types=[
            pltpu.VMEM((_N,), jnp.float32),
            pltpu.VMEM((2, _RC, _N), jnp.float32),
            pltpu.VMEM((2, _RC, _N), jnp.float32),
            pltpu.VMEM((rpw,), jnp.float32),
            pltpu.SemaphoreType.DMA((2,)),
            pltpu.SemaphoreType.DMA((2,)),
        ],
    )
    def sc_spikes(x_hbm, w_hbm, m_hbm, o_hbm, x_v, w_v, m_v, o_v, w_sem, m_sem):
        wid = lax.axis_index("s") * _NC + lax.axis_index("c")
        out0 = wid * rpw
        row0 = row_base + out0
        pltpu.sync_copy(x_hbm, x_v)

        def round_x(i, _):
            col = 16 * i
            x_v[pl.ds(col, 16)] = _round_bf16(x_v[pl.ds(col, 16)])
            return ()

        lax.fori_loop(0, _N // 16, round_x, (), unroll=8)

        def start(c, slot):
            pltpu.async_copy(
                w_hbm.at[pl.ds(row0 + c * _RC, _RC)], w_v.at[slot], w_sem.at[slot]
            )
            pltpu.async_copy(
                m_hbm.at[pl.ds(row0 + c * _RC, _RC)], m_v.at[slot], m_sem.at[slot]
            )

        lane = lax.broadcasted_iota(jnp.int32, (16,), 0)
        start(0, 0)
        start(1, 1)

        def pair_body(g, ovec):
            for b in range(2):
                c = 2 * g + b
                pltpu.make_async_copy(
                    w_hbm.at[pl.ds(0, _RC)], w_v.at[b], w_sem.at[b]
                ).wait()
                pltpu.make_async_copy(
                    m_hbm.at[pl.ds(0, _RC)], m_v.at[b], m_sem.at[b]
                ).wait()

                def k_body(k, accs):
                    col = 32 * k
                    xa = x_v[pl.ds(col, 16)]
                    xb = x_v[pl.ds(col + 16, 16)]
                    out = []
                    for r in range(_RC):
                        wa = w_v[b, r, pl.ds(col, 16)] * m_v[b, r, pl.ds(col, 16)]
                        wb = (
                            w_v[b, r, pl.ds(col + 16, 16)]
                            * m_v[b, r, pl.ds(col + 16, 16)]
                        )
                        out.append(
                            accs[r] + _round_bf16(wa) * xa + _round_bf16(wb) * xb
                        )
                    return tuple(out)

                accs = lax.fori_loop(
                    0,
                    _N // 32,
                    k_body,
                    tuple(jnp.zeros((16,), jnp.float32) for _ in range(_RC)),
                    unroll=2,
                )

                @pl.when(c + 2 < n_chunks)
                def _():
                    start_row = row0 + (c + 2) * _RC
                    pltpu.async_copy(
                        w_hbm.at[pl.ds(start_row, _RC)], w_v.at[b], w_sem.at[b]
                    )
                    pltpu.async_copy(
                        m_hbm.at[pl.ds(start_row, _RC)], m_v.at[b], m_sem.at[b]
                    )

                for r in range(_RC):
                    s = accs[r][0]
                    for i in range(1, 16):
                        s = s + accs[r][i]
                    spike = (s - _THRESHOLD >= 0.0).astype(jnp.float32)
                    lanepos = (c % 4) * _RC + r
                    ovec = jnp.where(lane == lanepos, spike, ovec)

            @pl.when(g % 2 == 1)
            def _():
                o_v[pl.ds((g // 2) * 16, 16)] = ovec

            return jnp.where(g % 2 == 1, jnp.zeros((16,), jnp.float32), ovec)

        lax.fori_loop(0, n_chunks // 2, pair_body, jnp.zeros((16,), jnp.float32))
        pltpu.sync_copy(o_v, o_hbm.at[pl.ds(out0, rpw)])

    return sc_spikes


_TC_BM = 256


def _tc_spike_kernel(x_hbm, w_ref, m_ref, o_ref, xs_ref, sem):
    @pl.when(pl.program_id(0) == 0)
    def _():
        cp = pltpu.make_async_copy(x_hbm, xs_ref, sem)
        cp.start()
        cp.wait()

    wm = (w_ref[...] * m_ref[...]).astype(jnp.bfloat16)
    xv = xs_ref[...].astype(jnp.bfloat16)
    acc = jax.lax.dot_general(
        wm, xv, (((1,), (0,)), ((), ())), preferred_element_type=jnp.float32
    )
    o_ref[...] = (acc - _THRESHOLD >= 0.0).astype(jnp.float32)


def _tc_spikes(x, weight, weight_mask, n_rows):
    return pl.pallas_call(
        _tc_spike_kernel,
        grid=(n_rows // _TC_BM,),
        in_specs=[
            pl.BlockSpec(memory_space=pl.ANY),
            pl.BlockSpec((_TC_BM, _N), lambda i: (i, 0)),
            pl.BlockSpec((_TC_BM, _N), lambda i: (i, 0)),
        ],
        out_specs=pl.BlockSpec((_TC_BM,), lambda i: (i,)),
        out_shape=jax.ShapeDtypeStruct((n_rows,), jnp.float32),
        scratch_shapes=[
            pltpu.VMEM((_N,), jnp.float32),
            pltpu.SemaphoreType.DMA,
        ],
    )(x, weight, weight_mask)


_MR = 64
_NBUF = 4


def _tc3_kernel(x_hbm, w_hbm, m_hbm, o_ref, xs_ref, wbuf, mbuf, xsem, wsem, msem):
    cpx = pltpu.make_async_copy(x_hbm, xs_ref, xsem)
    cpx.start()

    def start(c, slot):
        pltpu.make_async_copy(
            w_hbm.at[pl.ds(c * _MR, _MR)], wbuf.at[slot], wsem.at[slot]
        ).start()
        pltpu.make_async_copy(
            m_hbm.at[pl.ds(c * _MR, _MR)], mbuf.at[slot], msem.at[slot]
        ).start()

    n_chunks = _N // _MR
    for s in range(_NBUF):
        start(s, s)
    cpx.wait()
    xv = xs_ref[...].astype(jnp.bfloat16)
    for c in range(n_chunks):
        slot = c % _NBUF
        pltpu.make_async_copy(
            w_hbm.at[pl.ds(c * _MR, _MR)], wbuf.at[slot], wsem.at[slot]
        ).wait()
        pltpu.make_async_copy(
            m_hbm.at[pl.ds(c * _MR, _MR)], mbuf.at[slot], msem.at[slot]
        ).wait()
        wm = (wbuf[slot] * mbuf[slot]).astype(jnp.bfloat16)
        acc = jax.lax.dot_general(
            wm, xv, (((1,), (0,)), ((), ())), preferred_element_type=jnp.float32
        )
        o_ref[pl.ds(c * _MR, _MR)] = (acc - _THRESHOLD >= 0.0).astype(jnp.float32)
        if c + _NBUF < n_chunks:
            start(c + _NBUF, slot)


def _tc3_spikes(x, weight, weight_mask):
    return pl.pallas_call(
        _tc3_kernel,
        in_specs=[
            pl.BlockSpec(memory_space=pl.ANY),
            pl.BlockSpec(memory_space=pl.ANY),
            pl.BlockSpec(memory_space=pl.ANY),
        ],
        out_shape=jax.ShapeDtypeStruct((_N,), jnp.float32),
        scratch_shapes=[
            pltpu.VMEM((_N,), jnp.float32),
            pltpu.VMEM((_NBUF, _MR, _N), jnp.float32),
            pltpu.VMEM((_NBUF, _MR, _N), jnp.float32),
            pltpu.SemaphoreType.DMA,
            pltpu.SemaphoreType.DMA((_NBUF,)),
            pltpu.SemaphoreType.DMA((_NBUF,)),
        ],
    )(x, weight, weight_mask)


def kernel(x, weight, weight_mask):
    return _tc3_spikes(x, weight, weight_mask)
